# Initial kernel scaffold; baseline (speedup 1.0000x reference)
import jax
import jax.numpy as jnp
from jax.experimental import pallas as pl


def kernel(x, batch):
    raise NotImplementedError("write your pallas kernel here")


# speedup vs baseline: 4.0566x; 4.0566x over previous
import functools

import jax
import jax.numpy as jnp
from jax import lax
from jax.experimental import pallas as pl
from jax.experimental.pallas import tpu as pltpu
from jax.experimental.pallas import tpu_sc as plsc

N = 50000
D = 256
G = 128

NC = 2
NS = 16
L = 16
W = NC * NS
SPW = G // W
KD = D // L

CH = 128
CHD = CH * D


def _sc_body(x_hbm, b_hbm, out_hbm, batch_v, buf_v, acc_v):
    cid = lax.axis_index("c")
    sid = lax.axis_index("s")
    wid = sid * NC + cid

    pltpu.sync_copy(b_hbm, batch_v)

    lanes = lax.iota(jnp.int32, L)
    gtarg = jnp.minimum(wid * SPW + lanes, G)

    lo = jnp.zeros((L,), jnp.int32)
    hi = jnp.full((L,), N, jnp.int32)
    for _ in range(17):
        active = lo < hi
        mid = (lo + hi) >> 1
        probe = plsc.load_gather(batch_v, [jnp.minimum(mid, N - 1)])
        pred = probe < gtarg
        lo = jnp.where(active & pred, mid + 1, lo)
        hi = jnp.where(active & jnp.logical_not(pred), mid, hi)

    def extract(i):
        return jnp.max(jnp.where(lanes == i, lo, jnp.int32(-1)))

    bounds = [extract(i) for i in range(SPW + 1)]
    s0 = bounds[0]
    s_end = bounds[SPW]

    neg_inf = jnp.full((L,), -jnp.inf, jnp.float32)
    for k in range(SPW * KD):
        acc_v[pl.ds(L * k, L)] = neg_inf

    nch = (s_end - s0 + CH - 1) // CH

    def chunk_body(cc, carry):
        off = jnp.minimum(s0 + cc * CH, N - CH)
        pltpu.sync_copy(x_hbm.at[pl.ds(off * D, CHD)], buf_v)
        for gi in range(SPW):
            j_lo = jnp.clip(bounds[gi] - off, 0, CH)
            j_hi = jnp.clip(bounds[gi + 1] - off, 0, CH)
            accs = tuple(acc_v[pl.ds(gi * D + L * k, L)] for k in range(KD))

            def row_body(j, accs):
                base = j * D
                return tuple(
                    jnp.maximum(accs[k], buf_v[pl.ds(base + L * k, L)])
                    for k in range(KD))

            accs = lax.fori_loop(j_lo, j_hi, row_body, accs)
            for k in range(KD):
                acc_v[pl.ds(gi * D + L * k, L)] = accs[k]
        return carry

    lax.fori_loop(0, nch, chunk_body, jnp.int32(0))

    pltpu.sync_copy(acc_v, out_hbm.at[pl.ds(wid * SPW * D, SPW * D)])


@jax.jit
def _sc_segment_max(x_flat, batch):
    mesh = plsc.VectorSubcoreMesh(core_axis_name="c", subcore_axis_name="s")
    return pl.kernel(
        _sc_body,
        out_type=jax.ShapeDtypeStruct((G * D,), jnp.float32),
        mesh=mesh,
        compiler_params=pltpu.CompilerParams(needs_layout_passes=False),
        scratch_types=[
            pltpu.VMEM((N,), jnp.int32),
            pltpu.VMEM((CHD,), jnp.float32),
            pltpu.VMEM((SPW * D,), jnp.float32),
        ],
    )(x_flat, batch)


def kernel(x, batch):
    out = _sc_segment_max(x.reshape(-1), batch)
    return out.reshape(G, D)


# speedup vs baseline: 4.6191x; 1.1387x over previous
import functools

import jax
import jax.numpy as jnp
from jax import lax
from jax.experimental import pallas as pl
from jax.experimental.pallas import tpu as pltpu
from jax.experimental.pallas import tpu_sc as plsc

N = 50000
D = 256
G = 128

NC = 2
NS = 16
L = 16
W = NC * NS
SPW = G // W
KD = D // L

CH = 128
CHD = CH * D


def _sc_body(x_hbm, b_hbm, out_hbm, batch_v, buf0_v, buf1_v, acc_v,
             sem0, sem1):
    cid = lax.axis_index("c")
    sid = lax.axis_index("s")
    wid = sid * NC + cid

    pltpu.sync_copy(b_hbm, batch_v)

    lanes = lax.iota(jnp.int32, L)
    gtarg = jnp.minimum(wid * SPW + lanes, G)

    lo = jnp.zeros((L,), jnp.int32)
    hi = jnp.full((L,), N, jnp.int32)
    for _ in range(17):
        active = lo < hi
        mid = (lo + hi) >> 1
        probe = plsc.load_gather(batch_v, [jnp.minimum(mid, N - 1)])
        pred = probe < gtarg
        lo = jnp.where(active & pred, mid + 1, lo)
        hi = jnp.where(active & jnp.logical_not(pred), mid, hi)

    def extract(i):
        return jnp.max(jnp.where(lanes == i, lo, jnp.int32(-1)))

    bounds = [extract(i) for i in range(SPW + 1)]
    s0 = bounds[0]
    s_end = bounds[SPW]

    neg_inf = jnp.full((L,), -jnp.inf, jnp.float32)
    for k in range(SPW * KD):
        acc_v[pl.ds(L * k, L)] = neg_inf

    nch = (s_end - s0 + CH - 1) // CH
    bufs = (buf0_v, buf1_v)
    sems = (sem0, sem1)

    def issue(slot, cc):
        off = jnp.minimum(s0 + cc * CH, N - CH)
        pltpu.async_copy(x_hbm.at[pl.ds(off * D, CHD)], bufs[slot],
                         sems[slot])

    def wait(slot):
        pltpu.make_async_copy(x_hbm.at[pl.ds(0, CHD)], bufs[slot],
                              sems[slot]).wait()

    def process(slot, cc, valid):
        buf = bufs[slot]
        off = jnp.minimum(s0 + cc * CH, N - CH)
        for gi in range(SPW):
            j_lo = jnp.clip(bounds[gi] - off, 0, CH)
            j_hi = jnp.where(valid, jnp.clip(bounds[gi + 1] - off, 0, CH), 0)
            accs = tuple(acc_v[pl.ds(gi * D + L * k, L)] for k in range(KD))

            def row_body(j, accs):
                base = j * D
                return tuple(
                    jnp.maximum(accs[k], buf[pl.ds(base + L * k, L)])
                    for k in range(KD))

            accs = lax.fori_loop(j_lo, j_hi, row_body, accs)
            for k in range(KD):
                acc_v[pl.ds(gi * D + L * k, L)] = accs[k]

    @pl.when(nch > 0)
    def _():
        issue(0, 0)

    def pair_body(p, carry):
        cc0 = 2 * p
        cc1 = cc0 + 1
        wait(0)

        @pl.when(cc1 < nch)
        def _():
            issue(1, cc1)

        process(0, cc0, True)

        @pl.when(cc1 < nch)
        def _():
            wait(1)

            @pl.when(cc1 + 1 < nch)
            def _():
                issue(0, cc1 + 1)

        process(1, cc1, cc1 < nch)
        return carry

    lax.fori_loop(0, (nch + 1) >> 1, pair_body, jnp.int32(0))

    pltpu.sync_copy(acc_v, out_hbm.at[pl.ds(wid * SPW * D, SPW * D)])


@jax.jit
def _sc_segment_max(x_flat, batch):
    mesh = plsc.VectorSubcoreMesh(core_axis_name="c", subcore_axis_name="s")
    return pl.kernel(
        _sc_body,
        out_type=jax.ShapeDtypeStruct((G * D,), jnp.float32),
        mesh=mesh,
        compiler_params=pltpu.CompilerParams(needs_layout_passes=False),
        scratch_types=[
            pltpu.VMEM((N,), jnp.int32),
            pltpu.VMEM((CHD,), jnp.float32),
            pltpu.VMEM((CHD,), jnp.float32),
            pltpu.VMEM((SPW * D,), jnp.float32),
            pltpu.SemaphoreType.DMA,
            pltpu.SemaphoreType.DMA,
        ],
    )(x_flat, batch)


def kernel(x, batch):
    out = _sc_segment_max(x.reshape(-1), batch)
    return out.reshape(G, D)


# speedup vs baseline: 7.9454x; 1.7201x over previous
import jax
import jax.numpy as jnp
from jax import lax
from jax.experimental import pallas as pl
from jax.experimental.pallas import tpu as pltpu
from jax.experimental.pallas import tpu_sc as plsc

N = 50000
D = 256
G = 128

NC = 2
NS = 16
L = 16
W = NC * NS
SPW = G // W
KD = D // L

CH = 128


def _sc_body(x_hbm, b_hbm, out_hbm, batch_v, buf0_v, buf1_v, acc_v,
             sem0, sem1):
    cid = lax.axis_index("c")
    sid = lax.axis_index("s")
    wid = sid * NC + cid

    pltpu.sync_copy(b_hbm, batch_v)

    lanes = lax.iota(jnp.int32, L)
    gtarg = jnp.minimum(wid * SPW + lanes, G)

    lo = jnp.zeros((L,), jnp.int32)
    hi = jnp.full((L,), N, jnp.int32)
    for _ in range(17):
        active = lo < hi
        mid = (lo + hi) >> 1
        probe = plsc.load_gather(batch_v, [jnp.minimum(mid, N - 1)])
        pred = probe < gtarg
        lo = jnp.where(active & pred, mid + 1, lo)
        hi = jnp.where(active & jnp.logical_not(pred), mid, hi)

    def extract(i):
        return jnp.max(jnp.where(lanes == i, lo, jnp.int32(-1)))

    bounds = [extract(i) for i in range(SPW + 1)]
    s0 = bounds[0]
    s_end = bounds[SPW]

    neg_inf = jnp.full((L,), -jnp.inf, jnp.float32)
    for k in range(SPW * KD):
        acc_v[pl.ds(L * k, L)] = neg_inf

    a0 = (s0 >> 3) << 3
    nch = (s_end - a0 + CH - 1) // CH
    bufs = (buf0_v, buf1_v)
    sems = (sem0, sem1)

    def chunk_off(cc):
        return pl.multiple_of(jnp.minimum(a0 + cc * CH, N - CH), 8)

    def issue(slot, cc):
        pltpu.async_copy(x_hbm.at[pl.ds(chunk_off(cc), CH)], bufs[slot],
                         sems[slot])

    def wait(slot):
        pltpu.make_async_copy(x_hbm.at[pl.ds(0, CH)], bufs[slot],
                              sems[slot]).wait()

    def process(slot, cc, valid):
        buf = bufs[slot]
        off = chunk_off(cc)
        for gi in range(SPW):
            j_lo = jnp.clip(bounds[gi] - off, 0, CH)
            j_hi = jnp.where(valid, jnp.clip(bounds[gi + 1] - off, 0, CH), 0)
            accs = tuple(acc_v[pl.ds(gi * D + L * k, L)] for k in range(KD))

            def row_body(j, accs):
                return tuple(
                    jnp.maximum(accs[k], buf[j, pl.ds(L * k, L)])
                    for k in range(KD))

            accs = lax.fori_loop(j_lo, j_hi, row_body, accs)
            for k in range(KD):
                acc_v[pl.ds(gi * D + L * k, L)] = accs[k]

    @pl.when(nch > 0)
    def _():
        issue(0, 0)

    def pair_body(p, carry):
        cc0 = 2 * p
        cc1 = cc0 + 1
        wait(0)

        @pl.when(cc1 < nch)
        def _():
            issue(1, cc1)

        process(0, cc0, True)

        @pl.when(cc1 < nch)
        def _():
            wait(1)

            @pl.when(cc1 + 1 < nch)
            def _():
                issue(0, cc1 + 1)

        process(1, cc1, cc1 < nch)
        return carry

    lax.fori_loop(0, (nch + 1) >> 1, pair_body, jnp.int32(0))

    pltpu.sync_copy(acc_v, out_hbm.at[pl.ds(wid * SPW * D, SPW * D)])


@jax.jit
def _sc_segment_max(x, batch):
    mesh = plsc.VectorSubcoreMesh(core_axis_name="c", subcore_axis_name="s")
    return pl.kernel(
        _sc_body,
        out_type=jax.ShapeDtypeStruct((G * D,), jnp.float32),
        mesh=mesh,
        compiler_params=pltpu.CompilerParams(needs_layout_passes=False,
                                             use_tc_tiling_on_sc=True),
        scratch_types=[
            pltpu.VMEM((N,), jnp.int32),
            pltpu.VMEM((CH, D), jnp.float32),
            pltpu.VMEM((CH, D), jnp.float32),
            pltpu.VMEM((SPW * D,), jnp.float32),
            pltpu.SemaphoreType.DMA,
            pltpu.SemaphoreType.DMA,
        ],
    )(x, batch)


def kernel(x, batch):
    out = _sc_segment_max(x, batch)
    return out.reshape(G, D)


# speedup vs baseline: 8.0624x; 1.0147x over previous
import jax
import jax.numpy as jnp
from jax import lax
from jax.experimental import pallas as pl
from jax.experimental.pallas import tpu as pltpu
from jax.experimental.pallas import tpu_sc as plsc

N = 50000
D = 256
G = 128

NC = 2
NS = 16
L = 16
W = NC * NS
SPW = G // W
KD = D // L

CH = 128


def _sc_body(x_hbm, b_hbm, out_hbm, batch_v, buf_v, acc_v, sems):
    cid = lax.axis_index("c")
    sid = lax.axis_index("s")
    wid = sid * NC + cid

    pltpu.sync_copy(b_hbm, batch_v)

    lanes = lax.iota(jnp.int32, L)
    gtarg = jnp.minimum(wid * SPW + lanes, G)

    lo = jnp.zeros((L,), jnp.int32)
    hi = jnp.full((L,), N, jnp.int32)
    for _ in range(17):
        active = lo < hi
        mid = (lo + hi) >> 1
        probe = plsc.load_gather(batch_v, [jnp.minimum(mid, N - 1)])
        pred = probe < gtarg
        lo = jnp.where(active & pred, mid + 1, lo)
        hi = jnp.where(active & jnp.logical_not(pred), mid, hi)

    def extract(i):
        return jnp.max(jnp.where(lanes == i, lo, jnp.int32(-1)))

    bounds = [extract(i) for i in range(SPW + 1)]
    s0 = bounds[0]
    s_end = bounds[SPW]

    neg_inf = jnp.full((L,), -jnp.inf, jnp.float32)
    for k in range(SPW * KD):
        acc_v[pl.ds(L * k, L)] = neg_inf

    a0 = (s0 >> 3) << 3
    nch = (s_end - a0 + CH - 1) // CH

    def chunk_off(cc):
        return pl.multiple_of(jnp.minimum(a0 + cc * CH, N - CH), 8)

    def slot_base(cc):
        return pl.multiple_of((cc & 1) * CH, 8)

    def issue(cc):
        pltpu.async_copy(x_hbm.at[pl.ds(chunk_off(cc), CH)],
                         buf_v.at[pl.ds(slot_base(cc), CH)],
                         sems.at[cc & 1])

    def wait(cc):
        pltpu.make_async_copy(x_hbm.at[pl.ds(0, CH)],
                              buf_v.at[pl.ds(slot_base(cc), CH)],
                              sems.at[cc & 1]).wait()

    @pl.when(nch > 0)
    def _():
        issue(0)

    def chunk_body(cc, carry):
        wait(cc)

        @pl.when(cc + 1 < nch)
        def _():
            issue(cc + 1)

        base = slot_base(cc)
        off = chunk_off(cc)
        for gi in range(SPW):
            j_lo = jnp.clip(bounds[gi] - off, 0, CH)
            j_hi = jnp.clip(bounds[gi + 1] - off, 0, CH)
            accs = tuple(acc_v[pl.ds(gi * D + L * k, L)] for k in range(KD))

            def row_body(j, accs):
                return tuple(
                    jnp.maximum(accs[k], buf_v[base + j, pl.ds(L * k, L)])
                    for k in range(KD))

            accs = lax.fori_loop(j_lo, j_hi, row_body, accs)
            for k in range(KD):
                acc_v[pl.ds(gi * D + L * k, L)] = accs[k]
        return carry

    lax.fori_loop(0, nch, chunk_body, jnp.int32(0))

    pltpu.sync_copy(acc_v, out_hbm.at[pl.ds(wid * SPW * D, SPW * D)])


@jax.jit
def _sc_segment_max(x, batch):
    mesh = plsc.VectorSubcoreMesh(core_axis_name="c", subcore_axis_name="s")
    return pl.kernel(
        _sc_body,
        out_type=jax.ShapeDtypeStruct((G * D,), jnp.float32),
        mesh=mesh,
        compiler_params=pltpu.CompilerParams(needs_layout_passes=False,
                                             use_tc_tiling_on_sc=True),
        scratch_types=[
            pltpu.VMEM((N,), jnp.int32),
            pltpu.VMEM((2 * CH, D), jnp.float32),
            pltpu.VMEM((SPW * D,), jnp.float32),
            pltpu.SemaphoreType.DMA((2,)),
        ],
    )(x, batch)


def kernel(x, batch):
    out = _sc_segment_max(x, batch)
    return out.reshape(G, D)


# speedup vs baseline: 8.5922x; 1.0657x over previous
import jax
import jax.numpy as jnp
from jax import lax
from jax.experimental import pallas as pl
from jax.experimental.pallas import tpu as pltpu
from jax.experimental.pallas import tpu_sc as plsc

N = 50000
D = 256
G = 128

NC = 2
NS = 16
L = 16
W = NC * NS
SPW = G // W
KD = D // L

CH = 128


def _sc_body(x_hbm, b_hbm, out_hbm, batch_sh, batch_v, buf_v, acc_v, sems):
    cid = lax.axis_index("c")
    sid = lax.axis_index("s")
    wid = sid * NC + cid

    @pl.when(sid == 0)
    def _():
        pltpu.sync_copy(b_hbm, batch_sh)

    plsc.subcore_barrier()
    pltpu.sync_copy(batch_sh, batch_v)

    lanes = lax.iota(jnp.int32, L)
    gtarg = jnp.minimum(wid * SPW + lanes, G)

    def bs_body(_, lohi):
        lo, hi = lohi
        active = lo < hi
        mid = (lo + hi) >> 1
        probe = plsc.load_gather(batch_v, [jnp.minimum(mid, N - 1)])
        pred = probe < gtarg
        lo = jnp.where(active & pred, mid + 1, lo)
        hi = jnp.where(active & jnp.logical_not(pred), mid, hi)
        return lo, hi

    lo, _ = lax.fori_loop(
        0, 17, bs_body,
        (jnp.zeros((L,), jnp.int32), jnp.full((L,), N, jnp.int32)))

    def extract(i):
        return jnp.max(jnp.where(lanes == i, lo, jnp.int32(-1)))

    bounds = [extract(i) for i in range(SPW + 1)]
    s0 = bounds[0]
    s_end = bounds[SPW]

    neg_inf = jnp.full((L,), -jnp.inf, jnp.float32)

    def init_body(k, carry):
        acc_v[pl.ds(pl.multiple_of(L * k, L), L)] = neg_inf
        return carry

    lax.fori_loop(0, SPW * KD, init_body, jnp.int32(0))

    a0 = (s0 >> 3) << 3
    nch = (s_end - a0 + CH - 1) // CH

    def chunk_off(cc):
        return pl.multiple_of(jnp.minimum(a0 + cc * CH, N - CH), 8)

    def slot_base(cc):
        return pl.multiple_of((cc & 1) * CH, 8)

    def issue(cc):
        pltpu.async_copy(x_hbm.at[pl.ds(chunk_off(cc), CH)],
                         buf_v.at[pl.ds(slot_base(cc), CH)],
                         sems.at[cc & 1])

    def wait(cc):
        pltpu.make_async_copy(x_hbm.at[pl.ds(0, CH)],
                              buf_v.at[pl.ds(slot_base(cc), CH)],
                              sems.at[cc & 1]).wait()

    @pl.when(nch > 0)
    def _():
        issue(0)

    def chunk_body(cc, carry):
        wait(cc)

        @pl.when(cc + 1 < nch)
        def _():
            issue(cc + 1)

        base = slot_base(cc)
        off = chunk_off(cc)
        for gi in range(SPW):
            j_lo = jnp.clip(bounds[gi] - off, 0, CH)
            j_hi = jnp.clip(bounds[gi + 1] - off, 0, CH)
            accs = tuple(acc_v[pl.ds(gi * D + L * k, L)] for k in range(KD))

            def row_body(j, accs):
                return tuple(
                    jnp.maximum(accs[k], buf_v[base + j, pl.ds(L * k, L)])
                    for k in range(KD))

            accs = lax.fori_loop(j_lo, j_hi, row_body, accs)
            for k in range(KD):
                acc_v[pl.ds(gi * D + L * k, L)] = accs[k]
        return carry

    lax.fori_loop(0, nch, chunk_body, jnp.int32(0))

    pltpu.sync_copy(acc_v, out_hbm.at[pl.ds(wid * SPW * D, SPW * D)])


@jax.jit
def _sc_segment_max(x, batch):
    mesh = plsc.VectorSubcoreMesh(core_axis_name="c", subcore_axis_name="s")
    return pl.kernel(
        _sc_body,
        out_type=jax.ShapeDtypeStruct((G * D,), jnp.float32),
        mesh=mesh,
        compiler_params=pltpu.CompilerParams(needs_layout_passes=False,
                                             use_tc_tiling_on_sc=True),
        scratch_types=[
            pltpu.VMEM_SHARED((N,), jnp.int32),
            pltpu.VMEM((N,), jnp.int32),
            pltpu.VMEM((2 * CH, D), jnp.float32),
            pltpu.VMEM((SPW * D,), jnp.float32),
            pltpu.SemaphoreType.DMA((2,)),
        ],
    )(x, batch)


def kernel(x, batch):
    out = _sc_segment_max(x, batch)
    return out.reshape(G, D)


# speedup vs baseline: 8.6349x; 1.0050x over previous
import jax
import jax.numpy as jnp
from jax import lax
from jax.experimental import pallas as pl
from jax.experimental.pallas import tpu as pltpu
from jax.experimental.pallas import tpu_sc as plsc

N = 50000
D = 256
G = 128

NC = 2
NS = 16
L = 16
W = NC * NS
SPW = G // W
KD = D // L

CH = 128


def _sc_body(x_hbm, b_hbm, out_hbm, batch_sh, batch_v, buf_v, acc_v, sems):
    cid = lax.axis_index("c")
    sid = lax.axis_index("s")
    wid = sid * NC + cid

    @pl.when(sid == 0)
    def _():
        pltpu.sync_copy(b_hbm, batch_sh)

    plsc.subcore_barrier()
    pltpu.sync_copy(batch_sh, batch_v)

    lanes = lax.iota(jnp.int32, L)
    gtarg = jnp.minimum(wid * SPW + lanes, G)

    def bs_body(_, lohi):
        lo, hi = lohi
        active = lo < hi
        mid = (lo + hi) >> 1
        probe = plsc.load_gather(batch_v, [jnp.minimum(mid, N - 1)])
        pred = probe < gtarg
        lo = jnp.where(active & pred, mid + 1, lo)
        hi = jnp.where(active & jnp.logical_not(pred), mid, hi)
        return lo, hi

    lo, _ = lax.fori_loop(
        0, 17, bs_body,
        (jnp.zeros((L,), jnp.int32), jnp.full((L,), N, jnp.int32)))

    def extract(i):
        return jnp.max(jnp.where(lanes == i, lo, jnp.int32(-1)))

    s0 = extract(0)
    s_end = extract(SPW)

    neg_inf = jnp.full((L,), -jnp.inf, jnp.float32)

    def init_body(k, carry):
        acc_v[pl.ds(pl.multiple_of(L * k, L), L)] = neg_inf
        return carry

    lax.fori_loop(0, SPW * KD, init_body, jnp.int32(0))

    a0 = (s0 >> 3) << 3
    nch = (s_end - a0 + CH - 1) // CH

    def chunk_off(cc):
        return pl.multiple_of(jnp.minimum(a0 + cc * CH, N - CH), 8)

    def slot_base(cc):
        return pl.multiple_of((cc & 1) * CH, 8)

    def issue(cc):
        pltpu.async_copy(x_hbm.at[pl.ds(chunk_off(cc), CH)],
                         buf_v.at[pl.ds(slot_base(cc), CH)],
                         sems.at[cc & 1])

    def wait(cc):
        pltpu.make_async_copy(x_hbm.at[pl.ds(0, CH)],
                              buf_v.at[pl.ds(slot_base(cc), CH)],
                              sems.at[cc & 1]).wait()

    @pl.when(nch > 0)
    def _():
        issue(0)

    def chunk_body(cc, carry):
        wait(cc)

        @pl.when(cc + 1 < nch)
        def _():
            issue(cc + 1)

        base = slot_base(cc)
        off = chunk_off(cc)

        def seg_body(gi, c2):
            b_lo = extract(gi)
            b_hi = extract(gi + 1)
            j_lo = jnp.clip(b_lo - off, 0, CH)
            j_hi = jnp.clip(b_hi - off, 0, CH)
            abase = pl.multiple_of(gi * D, L)
            accs = tuple(acc_v[pl.ds(abase + L * k, L)] for k in range(KD))

            def row_body(j, accs):
                return tuple(
                    jnp.maximum(accs[k], buf_v[base + j, pl.ds(L * k, L)])
                    for k in range(KD))

            accs = lax.fori_loop(j_lo, j_hi, row_body, accs)
            for k in range(KD):
                acc_v[pl.ds(abase + L * k, L)] = accs[k]
            return c2

        lax.fori_loop(0, SPW, seg_body, jnp.int32(0))
        return carry

    lax.fori_loop(0, nch, chunk_body, jnp.int32(0))

    pltpu.sync_copy(acc_v, out_hbm.at[pl.ds(wid * SPW * D, SPW * D)])


@jax.jit
def _sc_segment_max(x, batch):
    mesh = plsc.VectorSubcoreMesh(core_axis_name="c", subcore_axis_name="s")
    return pl.kernel(
        _sc_body,
        out_type=jax.ShapeDtypeStruct((G * D,), jnp.float32),
        mesh=mesh,
        compiler_params=pltpu.CompilerParams(needs_layout_passes=False,
                                             use_tc_tiling_on_sc=True),
        scratch_types=[
            pltpu.VMEM_SHARED((N,), jnp.int32),
            pltpu.VMEM((N,), jnp.int32),
            pltpu.VMEM((2 * CH, D), jnp.float32),
            pltpu.VMEM((SPW * D,), jnp.float32),
            pltpu.SemaphoreType.DMA((2,)),
        ],
    )(x, batch)


def kernel(x, batch):
    out = _sc_segment_max(x, batch)
    return out.reshape(G, D)
